# baseline (speedup 1.0000x reference)
import functools

import jax
import jax.numpy as jnp
from jax import lax
from jax.experimental import pallas as pl
from jax.experimental.pallas import tpu as pltpu
from jax.experimental.pallas import tpu_sc as plsc

_BB = 2048


def _scale_body(p_ref, mag_ref, temp_ref, u_ref, s_ref):
    p = jnp.clip(p_ref[0], 0.1, 0.9)
    mag = jnp.clip(mag_ref[0], 0.0, 2.0)
    t = temp_ref[0]
    logit_p = jnp.log(p) - jnp.log1p(-p)
    u = u_ref[...]
    logistic = jnp.log(u) - jnp.log1p(-u)
    mask = jax.nn.sigmoid((logit_p + logistic) / t)
    s_ref[...] = 1.0 + mask * mag


def _blend_body(s_ref, e_ref, o_ref):
    o_ref[...] = e_ref[...] * s_ref[...]


def _sc_copy_body(rows_per_worker, nc, bts_hbm, out_hbm, buf):
    wid = lax.axis_index("s") * nc + lax.axis_index("c")
    base = wid * rows_per_worker
    pltpu.sync_copy(bts_hbm.at[pl.ds(base, rows_per_worker)], buf)
    pltpu.sync_copy(buf, out_hbm.at[pl.ds(base, rows_per_worker)])


def _sc_copy(bts):
    B, L = bts.shape
    info = plsc.get_sparse_core_info()
    nw = info.num_cores * info.num_subcores
    rpw = B // nw
    mesh = plsc.VectorSubcoreMesh(core_axis_name="c", subcore_axis_name="s")
    return pl.kernel(
        functools.partial(_sc_copy_body, rpw, info.num_cores),
        mesh=mesh,
        out_type=jax.ShapeDtypeStruct((B, L), jnp.int32),
        scratch_types=[pltpu.VMEM((rpw, L), jnp.int32)],
    )(bts)


def kernel(args, input, embed, labels, bts, ctx, eda, model, p_param, mag_param, temperature):
    B, D = embed.shape
    u = jax.random.uniform(
        jax.random.key(42), (B // 128, 128), minval=1e-6, maxval=1.0 - 1e-6,
        dtype=jnp.float32,
    )
    s2d = pl.pallas_call(
        _scale_body,
        in_specs=[
            pl.BlockSpec(memory_space=pltpu.SMEM),
            pl.BlockSpec(memory_space=pltpu.SMEM),
            pl.BlockSpec(memory_space=pltpu.SMEM),
            pl.BlockSpec((B // 128, 128), lambda: (0, 0)),
        ],
        out_specs=pl.BlockSpec((B // 128, 128), lambda: (0, 0)),
        out_shape=jax.ShapeDtypeStruct((B // 128, 128), jnp.float32),
    )(p_param, mag_param, temperature, u)
    s = s2d.reshape(B, 1)
    bb = _BB if B % _BB == 0 else B
    out_embed = pl.pallas_call(
        _blend_body,
        grid=(B // bb,),
        in_specs=[
            pl.BlockSpec((bb, 1), lambda i: (i, 0)),
            pl.BlockSpec((bb, D), lambda i: (i, 0)),
        ],
        out_specs=pl.BlockSpec((bb, D), lambda i: (i, 0)),
        out_shape=jax.ShapeDtypeStruct((B, D), jnp.float32),
    )(s, embed)
    aug_input = _sc_copy(bts)
    return (aug_input, out_embed)


# speedup vs baseline: 1.0033x; 1.0033x over previous
import functools

import jax
import jax.numpy as jnp
from jax import lax
from jax.experimental import pallas as pl
from jax.experimental.pallas import tpu as pltpu
from jax.experimental.pallas import tpu_sc as plsc

_BB = 2048


def _scale_body(p_ref, mag_ref, temp_ref, u_ref, s_ref):
    p = jnp.clip(p_ref[0], 0.1, 0.9)
    mag = jnp.clip(mag_ref[0], 0.0, 2.0)
    t = temp_ref[0]
    logit_p = jnp.log(p) - jnp.log1p(-p)
    u = u_ref[...]
    logistic = jnp.log(u) - jnp.log1p(-u)
    mask = jax.nn.sigmoid((logit_p + logistic) / t)
    s_ref[...] = 1.0 + mask * mag


def _blend_body(s_ref, e_ref, o_ref):
    o_ref[...] = e_ref[...] * s_ref[...]


def _sc_copy_body(rows_per_worker, nc, bts_hbm, out_hbm, buf):
    wid = lax.axis_index("s") * nc + lax.axis_index("c")
    base = wid * rows_per_worker
    pltpu.sync_copy(bts_hbm.at[pl.ds(base, rows_per_worker)], buf)
    pltpu.sync_copy(buf, out_hbm.at[pl.ds(base, rows_per_worker)])


def _sc_copy(bts):
    B, L = bts.shape
    info = plsc.get_sparse_core_info()
    nw = info.num_cores * info.num_subcores
    rpw = B // nw
    mesh = plsc.VectorSubcoreMesh(core_axis_name="c", subcore_axis_name="s")
    return pl.kernel(
        functools.partial(_sc_copy_body, rpw, info.num_cores),
        mesh=mesh,
        out_type=jax.ShapeDtypeStruct((B, L), jnp.int32),
        scratch_types=[pltpu.VMEM((rpw, L), jnp.int32)],
    )(bts)


def kernel(args, input, embed, labels, bts, ctx, eda, model, p_param, mag_param, temperature):
    B, D = embed.shape
    u = jax.random.uniform(
        jax.random.key(42), (B // 128, 128), minval=1e-6, maxval=1.0 - 1e-6,
        dtype=jnp.float32,
    )
    aug_input = _sc_copy(bts)
    s2d = pl.pallas_call(
        _scale_body,
        in_specs=[
            pl.BlockSpec(memory_space=pltpu.SMEM),
            pl.BlockSpec(memory_space=pltpu.SMEM),
            pl.BlockSpec(memory_space=pltpu.SMEM),
            pl.BlockSpec((B // 128, 128), lambda: (0, 0)),
        ],
        out_specs=pl.BlockSpec((B // 128, 128), lambda: (0, 0)),
        out_shape=jax.ShapeDtypeStruct((B // 128, 128), jnp.float32),
    )(p_param, mag_param, temperature, u)
    s = s2d.reshape(B, 1)
    bb = _BB if B % _BB == 0 else B
    out_embed = pl.pallas_call(
        _blend_body,
        grid=(B // bb,),
        in_specs=[
            pl.BlockSpec((bb, 1), lambda i: (i, 0)),
            pl.BlockSpec((bb, D), lambda i: (i, 0)),
        ],
        out_specs=pl.BlockSpec((bb, D), lambda i: (i, 0)),
        out_shape=jax.ShapeDtypeStruct((B, D), jnp.float32),
    )(s, embed)
    return (aug_input, out_embed)


# speedup vs baseline: 1.2559x; 1.2518x over previous
import functools

import jax
import jax.numpy as jnp
from jax import lax
from jax.experimental import pallas as pl
from jax.experimental.pallas import tpu as pltpu
from jax.experimental.pallas import tpu_sc as plsc

_BB = 2048


def _scale_body(p_ref, mag_ref, temp_ref, u_ref, s_ref):
    p = jnp.clip(p_ref[0], 0.1, 0.9)
    mag = jnp.clip(mag_ref[0], 0.0, 2.0)
    t = temp_ref[0]
    logit_p = jnp.log(p) - jnp.log1p(-p)
    u = u_ref[...]
    logistic = jnp.log(u) - jnp.log1p(-u)
    mask = jax.nn.sigmoid((logit_p + logistic) / t)
    s = 1.0 + mask * mag
    n = s.shape[1]
    for i in range(s.shape[0]):
        s_ref[pl.ds(n * i, n), :] = s[i, :].reshape(n, 1)


def _blend_body(s_ref, e_ref, o_ref):
    o_ref[...] = e_ref[...] * s_ref[...]


def _sc_copy_body(rows_per_worker, nc, bts_hbm, out_hbm, buf):
    wid = lax.axis_index("s") * nc + lax.axis_index("c")
    base = wid * rows_per_worker
    pltpu.sync_copy(bts_hbm.at[pl.ds(base, rows_per_worker)], buf)
    pltpu.sync_copy(buf, out_hbm.at[pl.ds(base, rows_per_worker)])


def _sc_copy(bts):
    B, L = bts.shape
    info = plsc.get_sparse_core_info()
    nw = info.num_cores * info.num_subcores
    rpw = B // nw
    mesh = plsc.VectorSubcoreMesh(core_axis_name="c", subcore_axis_name="s")
    return pl.kernel(
        functools.partial(_sc_copy_body, rpw, info.num_cores),
        mesh=mesh,
        out_type=jax.ShapeDtypeStruct((B, L), jnp.int32),
        scratch_types=[pltpu.VMEM((rpw, L), jnp.int32)],
    )(bts)


def kernel(args, input, embed, labels, bts, ctx, eda, model, p_param, mag_param, temperature):
    B, D = embed.shape
    u = jax.random.uniform(
        jax.random.key(42), (B // 128, 128), minval=1e-6, maxval=1.0 - 1e-6,
        dtype=jnp.float32,
    )
    s = pl.pallas_call(
        _scale_body,
        in_specs=[
            pl.BlockSpec(memory_space=pltpu.SMEM),
            pl.BlockSpec(memory_space=pltpu.SMEM),
            pl.BlockSpec(memory_space=pltpu.SMEM),
            pl.BlockSpec((B // 128, 128), lambda: (0, 0)),
        ],
        out_specs=pl.BlockSpec((B, 1), lambda: (0, 0)),
        out_shape=jax.ShapeDtypeStruct((B, 1), jnp.float32),
    )(p_param, mag_param, temperature, u)
    bb = _BB if B % _BB == 0 else B
    out_embed = pl.pallas_call(
        _blend_body,
        grid=(B // bb,),
        in_specs=[
            pl.BlockSpec((bb, 1), lambda i: (i, 0)),
            pl.BlockSpec((bb, D), lambda i: (i, 0)),
        ],
        out_specs=pl.BlockSpec((bb, D), lambda i: (i, 0)),
        out_shape=jax.ShapeDtypeStruct((B, D), jnp.float32),
    )(s, embed)
    return (bts, out_embed)


# speedup vs baseline: 1.5398x; 1.2260x over previous
import jax
import jax.numpy as jnp
from jax.experimental import pallas as pl
from jax.experimental.pallas import tpu as pltpu

_BB = 2048


def _fused_body(p_ref, mag_ref, temp_ref, u_ref, e_ref, b_ref, ob_ref, o_ref):
    p = jnp.clip(p_ref[0], 0.1, 0.9)
    mag = jnp.clip(mag_ref[0], 0.0, 2.0)
    t = temp_ref[0]
    logit_p = jnp.log(p) - jnp.log1p(-p)
    u = u_ref[...]
    logistic = jnp.log(u) - jnp.log1p(-u)
    mask = jax.nn.sigmoid((logit_p + logistic) / t)
    s = 1.0 + mask * mag
    n = s.shape[1]
    scol = jnp.concatenate(
        [s[j, :].reshape(n, 1) for j in range(s.shape[0])], axis=0
    )
    o_ref[...] = e_ref[...] * scol
    ob_ref[...] = b_ref[...]


def kernel(args, input, embed, labels, bts, ctx, eda, model, p_param, mag_param, temperature):
    B, D = embed.shape
    L = bts.shape[1]
    u = jax.random.uniform(
        jax.random.key(42), (B // 128, 128), minval=1e-6, maxval=1.0 - 1e-6,
        dtype=jnp.float32,
    )
    bb = _BB
    nb = bb // 128
    out_bts, out_embed = pl.pallas_call(
        _fused_body,
        grid=(B // bb,),
        in_specs=[
            pl.BlockSpec(memory_space=pltpu.SMEM),
            pl.BlockSpec(memory_space=pltpu.SMEM),
            pl.BlockSpec(memory_space=pltpu.SMEM),
            pl.BlockSpec((nb, 128), lambda i: (i, 0)),
            pl.BlockSpec((bb, D), lambda i: (i, 0)),
            pl.BlockSpec((bb, L), lambda i: (i, 0)),
        ],
        out_specs=[
            pl.BlockSpec((bb, L), lambda i: (i, 0)),
            pl.BlockSpec((bb, D), lambda i: (i, 0)),
        ],
        out_shape=[
            jax.ShapeDtypeStruct((B, L), jnp.int32),
            jax.ShapeDtypeStruct((B, D), jnp.float32),
        ],
    )(p_param, mag_param, temperature, u, embed, bts)
    return (out_bts, out_embed)


# speedup vs baseline: 1.5772x; 1.0243x over previous
import jax
import jax.numpy as jnp
import numpy as np
from jax.experimental import pallas as pl
from jax.experimental.pallas import tpu as pltpu

_BB = 2048

_KS0 = np.uint32(0)
_KS1 = np.uint32(42)
_KS2 = np.uint32(_KS0 ^ _KS1 ^ np.uint32(0x1BD11BDA))
_ROTS = ((13, 15, 26, 6), (17, 29, 16, 24))
_INJECT = ((_KS1, _KS2, 1), (_KS2, _KS0, 2), (_KS0, _KS1, 3),
           (_KS1, _KS2, 4), (_KS2, _KS0, 5))


def _rotl(x, r):
    return (x << np.uint32(r)) | (x >> np.uint32(32 - r))


def _threefry_bits(idx):
    x0 = jnp.zeros_like(idx, dtype=jnp.uint32) + _KS0
    x1 = idx.astype(jnp.uint32) + _KS1
    for i in range(5):
        for r in _ROTS[i % 2]:
            x0 = x0 + x1
            x1 = _rotl(x1, r)
            x1 = x1 ^ x0
        a, b, c = _INJECT[i]
        x0 = x0 + a
        x1 = x1 + b + np.uint32(c)
    return x0 ^ x1


def _fused_body(p_ref, mag_ref, temp_ref, e_ref, b_ref, ob_ref, o_ref):
    nb = _BB // 128
    i = pl.program_id(0)
    idx = (
        i * _BB
        + jax.lax.broadcasted_iota(jnp.int32, (nb, 128), 0) * 128
        + jax.lax.broadcasted_iota(jnp.int32, (nb, 128), 1)
    )
    bits = _threefry_bits(idx)
    fbits = (bits >> np.uint32(9)) | np.uint32(0x3F800000)
    floats = jax.lax.bitcast_convert_type(fbits, jnp.float32) - 1.0
    minval = np.float32(1e-6)
    maxval = np.float32(1.0 - 1e-6)
    u = jnp.maximum(minval, floats * (maxval - minval) + minval)

    p = jnp.clip(p_ref[0], 0.1, 0.9)
    mag = jnp.clip(mag_ref[0], 0.0, 2.0)
    t = temp_ref[0]
    logit_p = jnp.log(p) - jnp.log1p(-p)
    logistic = jnp.log(u) - jnp.log1p(-u)
    mask = jax.nn.sigmoid((logit_p + logistic) / t)
    s = 1.0 + mask * mag
    scol = jnp.concatenate(
        [s[j, :].reshape(128, 1) for j in range(nb)], axis=0
    )
    o_ref[...] = e_ref[...] * scol
    ob_ref[...] = b_ref[...]


def kernel(args, input, embed, labels, bts, ctx, eda, model, p_param, mag_param, temperature):
    B, D = embed.shape
    L = bts.shape[1]
    bb = _BB
    out_bts, out_embed = pl.pallas_call(
        _fused_body,
        grid=(B // bb,),
        in_specs=[
            pl.BlockSpec(memory_space=pltpu.SMEM),
            pl.BlockSpec(memory_space=pltpu.SMEM),
            pl.BlockSpec(memory_space=pltpu.SMEM),
            pl.BlockSpec((bb, D), lambda i: (i, 0)),
            pl.BlockSpec((bb, L), lambda i: (i, 0)),
        ],
        out_specs=[
            pl.BlockSpec((bb, L), lambda i: (i, 0)),
            pl.BlockSpec((bb, D), lambda i: (i, 0)),
        ],
        out_shape=[
            jax.ShapeDtypeStruct((B, L), jnp.int32),
            jax.ShapeDtypeStruct((B, D), jnp.float32),
        ],
    )(p_param, mag_param, temperature, embed, bts)
    return (out_bts, out_embed)


# speedup vs baseline: 1.6511x; 1.0469x over previous
import jax
import jax.numpy as jnp
import numpy as np
from jax.experimental import pallas as pl
from jax.experimental.pallas import tpu as pltpu

_BB = 4096

_KS0 = np.uint32(0)
_KS1 = np.uint32(42)
_KS2 = np.uint32(_KS0 ^ _KS1 ^ np.uint32(0x1BD11BDA))
_ROTS = ((13, 15, 26, 6), (17, 29, 16, 24))
_INJECT = ((_KS1, _KS2, 1), (_KS2, _KS0, 2), (_KS0, _KS1, 3),
           (_KS1, _KS2, 4), (_KS2, _KS0, 5))


def _rotl(x, r):
    return (x << np.uint32(r)) | (x >> np.uint32(32 - r))


def _threefry_bits(idx):
    x0 = jnp.zeros_like(idx, dtype=jnp.uint32) + _KS0
    x1 = idx.astype(jnp.uint32) + _KS1
    for i in range(5):
        for r in _ROTS[i % 2]:
            x0 = x0 + x1
            x1 = _rotl(x1, r)
            x1 = x1 ^ x0
        a, b, c = _INJECT[i]
        x0 = x0 + a
        x1 = x1 + b + np.uint32(c)
    return x0 ^ x1


def _fused_body(p_ref, mag_ref, temp_ref, e_ref, b_ref, ob_ref, o_ref):
    nb = _BB // 128
    i = pl.program_id(0)
    idx = (
        i * _BB
        + jax.lax.broadcasted_iota(jnp.int32, (nb, 128), 0) * 128
        + jax.lax.broadcasted_iota(jnp.int32, (nb, 128), 1)
    )
    bits = _threefry_bits(idx)
    fbits = (bits >> np.uint32(9)) | np.uint32(0x3F800000)
    floats = jax.lax.bitcast_convert_type(fbits, jnp.float32) - 1.0
    minval = np.float32(1e-6)
    maxval = np.float32(1.0 - 1e-6)
    u = jnp.maximum(minval, floats * (maxval - minval) + minval)

    p = jnp.clip(p_ref[0], 0.1, 0.9)
    mag = jnp.clip(mag_ref[0], 0.0, 2.0)
    t = temp_ref[0]
    logit_p = jnp.log(p) - jnp.log1p(-p)
    logistic = jnp.log(u) - jnp.log1p(-u)
    mask = jax.nn.sigmoid((logit_p + logistic) / t)
    s = 1.0 + mask * mag
    scol = jnp.concatenate(
        [s[j, :].reshape(128, 1) for j in range(nb)], axis=0
    )
    o_ref[...] = e_ref[...] * scol
    ob_ref[...] = b_ref[...]


def kernel(args, input, embed, labels, bts, ctx, eda, model, p_param, mag_param, temperature):
    B, D = embed.shape
    L = bts.shape[1]
    bb = _BB
    out_bts, out_embed = pl.pallas_call(
        _fused_body,
        grid=(B // bb,),
        in_specs=[
            pl.BlockSpec(memory_space=pltpu.SMEM),
            pl.BlockSpec(memory_space=pltpu.SMEM),
            pl.BlockSpec(memory_space=pltpu.SMEM),
            pl.BlockSpec((bb, D), lambda i: (i, 0)),
            pl.BlockSpec((bb, L), lambda i: (i, 0)),
        ],
        out_specs=[
            pl.BlockSpec((bb, L), lambda i: (i, 0)),
            pl.BlockSpec((bb, D), lambda i: (i, 0)),
        ],
        out_shape=[
            jax.ShapeDtypeStruct((B, L), jnp.int32),
            jax.ShapeDtypeStruct((B, D), jnp.float32),
        ],
    )(p_param, mag_param, temperature, embed, bts)
    return (out_bts, out_embed)
